# Initial kernel scaffold; baseline (speedup 1.0000x reference)
import jax
import jax.numpy as jnp
from jax.experimental import pallas as pl


def kernel(node_feat, adj, fc1_w1, fc1_b1, fc1_w2, fc1_b2, fc1_w3, fc1_b3, fc2_w1, fc2_b1, fc2_w2, fc2_b2, fc2_w3, fc2_b3, weight_G, bias_G):
    raise NotImplementedError("write your pallas kernel here")


# speedup vs baseline: 10.3647x; 10.3647x over previous
import jax
import jax.numpy as jnp
from jax.experimental import pallas as pl
from jax.experimental.pallas import tpu as pltpu

_S = 112
_NS = 105
_QROWS = 16


def _meta_kernel(nf_ref, qcat_ref, wa_ref, wb_ref, b1_ref, w12_ref, b2_ref,
                 w13_ref, b3_ref, wg0_ref, wgq_ref, bg_ref,
                 w21_ref, b21_ref, w22_ref, b22_ref, w23_ref, b23_ref,
                 out_ref):
    nf = nf_ref[0]
    a_q = jnp.dot(nf[96:112], wa_ref[...],
                  preferred_element_type=jnp.float32)
    b_all = jnp.dot(nf, wb_ref[...],
                    preferred_element_type=jnp.float32)
    h1 = jax.nn.relu(a_q[:, None, :] + b_all[None, :, :]
                     + b1_ref[...][None])
    h2 = jax.nn.relu(jnp.dot(h1.reshape(_QROWS * _S, 256), w12_ref[...],
                             preferred_element_type=jnp.float32)
                     + b2_ref[...])
    s = jnp.sum(h2.reshape(_QROWS, _S, 128) * w13_ref[...][None], axis=-1)
    s = jax.nn.sigmoid(s + b3_ref[0, 0])
    col = jax.lax.broadcasted_iota(jnp.int32, (_QROWS, _S), 1)
    la_q = jnp.where(col < _NS, s, 0.0)
    sup = jnp.dot(nf, wg0_ref[...],
                  preferred_element_type=jnp.float32)
    qterm = jnp.dot(qcat_ref[0], wgq_ref[...],
                    preferred_element_type=jnp.float32)
    sup = sup + qterm
    wl = jax.nn.relu(jnp.dot(la_q, sup,
                             preferred_element_type=jnp.float32)
                     + bg_ref[...])
    g1 = jax.nn.relu(jnp.dot(wl, w21_ref[...],
                             preferred_element_type=jnp.float32)
                     + b21_ref[...])
    g2 = jax.nn.relu(jnp.dot(g1, w22_ref[...],
                             preferred_element_type=jnp.float32)
                     + b22_ref[...])
    out_ref[0] = jnp.dot(g2, w23_ref[...],
                         preferred_element_type=jnp.float32) + b23_ref[...]


def kernel(node_feat, adj, fc1_w1, fc1_b1, fc1_w2, fc1_b2, fc1_w3, fc1_b3,
           fc2_w1, fc2_b1, fc2_w2, fc2_b2, fc2_w3, fc2_b3, weight_G, bias_G):
    nt, ns, d = node_feat.shape
    nsup = adj.shape[1]
    del adj
    nq = ns - nsup
    nf = jnp.pad(node_feat, ((0, 0), (0, _S - ns), (0, 0)))
    qcat = node_feat[:, nsup:, :].reshape(nt, 1, nq * d)

    wa = fc1_w1[:, :d].T
    wb = fc1_w1[:, d:].T
    w12 = fc1_w2.T
    w13 = fc1_w3
    b3 = fc1_b3.reshape(1, 1)
    wg0 = weight_G[:d, :]
    wgq = weight_G[d:, :]
    w21 = fc2_w1.T
    w22 = fc2_w2.T
    w23 = jnp.pad(fc2_w3.T, ((0, 0), (0, 128 - nq)))
    b23 = jnp.pad(fc2_b3, (0, 128 - nq)).reshape(1, 128)
    b1 = fc1_b1.reshape(1, -1)
    b2 = fc1_b2.reshape(1, -1)
    bg = bias_G.reshape(1, -1)
    b21 = fc2_b1.reshape(1, -1)
    b22 = fc2_b2.reshape(1, -1)

    def task_map(t):
        return (t, 0, 0)

    def const2(t):
        return (0, 0)

    consts = [wa, wb, b1, w12, b2, w13, b3, wg0, wgq, bg,
              w21, b21, w22, b22, w23, b23]
    out = pl.pallas_call(
        _meta_kernel,
        grid=(nt,),
        in_specs=[pl.BlockSpec((1, _S, d), task_map),
                  pl.BlockSpec((1, 1, nq * d), task_map)]
                 + [pl.BlockSpec(c.shape, const2) for c in consts],
        out_specs=pl.BlockSpec((1, _QROWS, 128), task_map),
        out_shape=jax.ShapeDtypeStruct((nt, _QROWS, 128), jnp.float32),
        compiler_params=pltpu.CompilerParams(
            dimension_semantics=("arbitrary",)),
    )(nf, qcat, *consts)
    return out[:, 4:4 + nq, :nq]


# speedup vs baseline: 14.2769x; 1.3775x over previous
import jax
import jax.numpy as jnp
from jax.experimental import pallas as pl
from jax.experimental.pallas import tpu as pltpu

_S = 112
_NS = 105
_QROWS = 16

_DN_T = (((1,), (1,)), ((), ()))


def _dot_t(x, w):
    return jax.lax.dot_general(x, w, _DN_T,
                               preferred_element_type=jnp.float32)


def _meta_kernel(nf_ref, qcat_ref, w1_ref, b1_ref, w12_ref, b2_ref,
                 w13_ref, b3_ref, wg_ref, bg_ref,
                 w21_ref, b21_ref, w22_ref, b22_ref, w23_ref, b23_ref,
                 out_ref):
    nf = nf_ref[0]
    a_q = _dot_t(nf[96:112], w1_ref[:, 0:128])
    b_all = _dot_t(nf, w1_ref[:, 128:256])
    h1 = jax.nn.relu(a_q[:, None, :] + b_all[None, :, :]
                     + b1_ref[...][None, None, :])
    h2 = jax.nn.relu(_dot_t(h1.reshape(_QROWS * _S, 256), w12_ref[...])
                     + b2_ref[...][None, :])
    s = jnp.sum(h2.reshape(_QROWS, _S, 128) * w13_ref[...][None], axis=-1)
    s = jax.nn.sigmoid(s + b3_ref[0])
    col = jax.lax.broadcasted_iota(jnp.int32, (_QROWS, _S), 1)
    la_q = jnp.where(col < _NS, s, 0.0)
    sup = jnp.dot(nf, wg_ref[0:128, :],
                  preferred_element_type=jnp.float32)
    qterm = jnp.dot(qcat_ref[0], wg_ref[128:768, :],
                    preferred_element_type=jnp.float32)
    sup = sup + qterm
    wl = jax.nn.relu(jnp.dot(la_q, sup,
                             preferred_element_type=jnp.float32)
                     + bg_ref[...][None, :])
    g1 = jax.nn.relu(_dot_t(wl, w21_ref[...])
                     + b21_ref[...][None, :])
    g2 = jax.nn.relu(_dot_t(g1, w22_ref[...])
                     + b22_ref[...][None, :])
    out_ref[0] = _dot_t(g2, w23_ref[...]) + b23_ref[...][None, :]


def kernel(node_feat, adj, fc1_w1, fc1_b1, fc1_w2, fc1_b2, fc1_w3, fc1_b3,
           fc2_w1, fc2_b1, fc2_w2, fc2_b2, fc2_w3, fc2_b3, weight_G, bias_G):
    nt, ns, d = node_feat.shape
    nsup = adj.shape[1]
    del adj
    nq = ns - nsup
    nf = jnp.pad(node_feat, ((0, 0), (0, _S - ns), (0, 0)))
    qcat = node_feat[:, nsup:, :].reshape(nt, 1, nq * d)
    w23 = jnp.pad(fc2_w3, ((0, 128 - nq), (0, 0)))
    b23 = jnp.pad(fc2_b3, (0, 128 - nq))

    def task_map(t):
        return (t, 0, 0)

    consts = [fc1_w1, fc1_b1, fc1_w2, fc1_b2, fc1_w3, fc1_b3,
              weight_G, bias_G, fc2_w1, fc2_b1, fc2_w2, fc2_b2, w23, b23]

    def const_map_for(c):
        zeros = (0,) * c.ndim
        return lambda t: zeros

    out = pl.pallas_call(
        _meta_kernel,
        grid=(nt,),
        in_specs=[pl.BlockSpec((1, _S, d), task_map),
                  pl.BlockSpec((1, 1, nq * d), task_map)]
                 + [pl.BlockSpec(c.shape, const_map_for(c)) for c in consts],
        out_specs=pl.BlockSpec((1, _QROWS, 128), task_map),
        out_shape=jax.ShapeDtypeStruct((nt, _QROWS, 128), jnp.float32),
        compiler_params=pltpu.CompilerParams(
            dimension_semantics=("arbitrary",)),
    )(nf, qcat, *consts)
    return out[:, 4:4 + nq, :nq]


# speedup vs baseline: 16.1566x; 1.1317x over previous
import jax
import jax.numpy as jnp
from jax.experimental import pallas as pl
from jax.experimental.pallas import tpu as pltpu

_S = 112
_NS = 105
_SHIFT = 4
_QROWS = 8

_DN_T = (((1,), (1,)), ((), ()))


def _dot_t(x, w):
    return jax.lax.dot_general(x, w, _DN_T,
                               preferred_element_type=jnp.float32)


def _meta_kernel(nf_ref, qcat_ref, w1_ref, b1_ref, w12_ref, b2_ref,
                 w13_ref, b3_ref, wg_ref, bg_ref,
                 w21_ref, b21_ref, w22_ref, b22_ref, w23_ref, b23_ref,
                 out_ref):
    nf = nf_ref[0]
    a_q = _dot_t(nf[104:112], w1_ref[:, 0:128])
    b_all = _dot_t(nf, w1_ref[:, 128:256])
    h1 = jax.nn.relu(a_q[:, None, :] + b_all[None, :, :]
                     + b1_ref[...][None, None, :])
    h2 = jax.nn.relu(_dot_t(h1.reshape(_QROWS * _S, 256), w12_ref[...])
                     + b2_ref[...][None, :])
    s = jnp.sum(h2.reshape(_QROWS, _S, 128) * w13_ref[...][None], axis=-1)
    s = jax.nn.sigmoid(s + b3_ref[0])
    col = jax.lax.broadcasted_iota(jnp.int32, (_QROWS, _S), 1)
    la_q = jnp.where((col >= _SHIFT) & (col < _SHIFT + _NS), s, 0.0)
    sup = jnp.dot(nf, wg_ref[0:128, :],
                  preferred_element_type=jnp.float32)
    qterm = jnp.dot(qcat_ref[0], wg_ref[128:768, :],
                    preferred_element_type=jnp.float32)
    sup = sup + qterm
    wl = jax.nn.relu(jnp.dot(la_q, sup,
                             preferred_element_type=jnp.float32)
                     + bg_ref[...][None, :])
    g1 = jax.nn.relu(_dot_t(wl, w21_ref[...])
                     + b21_ref[...][None, :])
    g2 = jax.nn.relu(_dot_t(g1, w22_ref[...])
                     + b22_ref[...][None, :])
    out_ref[0] = _dot_t(g2, w23_ref[...]) + b23_ref[...][None, :]


def kernel(node_feat, adj, fc1_w1, fc1_b1, fc1_w2, fc1_b2, fc1_w3, fc1_b3,
           fc2_w1, fc2_b1, fc2_w2, fc2_b2, fc2_w3, fc2_b3, weight_G, bias_G):
    nt, ns, d = node_feat.shape
    nsup = adj.shape[1]
    del adj
    nq = ns - nsup
    nf = jnp.pad(node_feat, ((0, 0), (_SHIFT, _S - ns - _SHIFT), (0, 0)))
    qcat = node_feat[:, nsup:, :].reshape(nt, 1, nq * d)
    w23 = jnp.pad(fc2_w3, ((0, 128 - nq), (0, 0)))
    b23 = jnp.pad(fc2_b3, (0, 128 - nq))

    def task_map(t):
        return (t, 0, 0)

    consts = [fc1_w1, fc1_b1, fc1_w2, fc1_b2, fc1_w3, fc1_b3,
              weight_G, bias_G, fc2_w1, fc2_b1, fc2_w2, fc2_b2, w23, b23]

    def const_map_for(c):
        zeros = (0,) * c.ndim
        return lambda t: zeros

    out = pl.pallas_call(
        _meta_kernel,
        grid=(nt,),
        in_specs=[pl.BlockSpec((1, _S, d), task_map),
                  pl.BlockSpec((1, 1, nq * d), task_map)]
                 + [pl.BlockSpec(c.shape, const_map_for(c)) for c in consts],
        out_specs=pl.BlockSpec((1, _QROWS, 128), task_map),
        out_shape=jax.ShapeDtypeStruct((nt, _QROWS, 128), jnp.float32),
        compiler_params=pltpu.CompilerParams(
            dimension_semantics=("arbitrary",)),
    )(nf, qcat, *consts)
    return out[:, 0:nq, :nq]


# speedup vs baseline: 26.0844x; 1.6145x over previous
import jax
import jax.numpy as jnp
from jax.experimental import pallas as pl
from jax.experimental.pallas import tpu as pltpu

_S = 112
_NS = 105
_SHIFT = 4
_QROWS = 8
_T = 4

_DN_T = (((1,), (1,)), ((), ()))


def _dot_t(x, w):
    return jax.lax.dot_general(x, w, _DN_T,
                               preferred_element_type=jnp.float32)


def _meta_kernel(nf_ref, qcat_ref, w1_ref, b1_ref, w12_ref, b2_ref,
                 w13_ref, b3_ref, wg_ref, bg_ref,
                 w21_ref, b21_ref, w22_ref, b22_ref, w23_ref, b23_ref,
                 out_ref):
    nf3 = nf_ref[...]
    nf2 = nf3.reshape(_T * _S, 128)
    aq_in = nf3[:, _S - _QROWS:_S, :].reshape(_T * _QROWS, 128)
    a_q = _dot_t(aq_in, w1_ref[:, 0:128])
    b_all = _dot_t(nf2, w1_ref[:, 128:256])
    h1 = jax.nn.relu(a_q.reshape(_T, _QROWS, 1, 256)
                     + b_all.reshape(_T, 1, _S, 256)
                     + b1_ref[...][None, None, None, :])
    h2 = jax.nn.relu(_dot_t(h1.reshape(_T * _QROWS * _S, 256), w12_ref[...])
                     + b2_ref[...][None, :])
    s = jnp.sum(h2.reshape(_T, _QROWS, _S, 128)
                * w13_ref[...][None, None], axis=-1)
    s = jax.nn.sigmoid(s + b3_ref[0])
    col = jax.lax.broadcasted_iota(jnp.int32, (_T, _QROWS, _S), 2)
    la3 = jnp.where((col >= _SHIFT) & (col < _SHIFT + _NS), s, 0.0)
    sup = jnp.dot(nf2, wg_ref[0:128, :],
                  preferred_element_type=jnp.float32)
    qterm = jnp.dot(qcat_ref[...].reshape(_T, 640), wg_ref[128:768, :],
                    preferred_element_type=jnp.float32)
    sup3 = sup.reshape(_T, _S, 768) + qterm[:, None, :]
    wl = jax.lax.dot_general(la3, sup3, (((2,), (1,)), ((0,), (0,))),
                             preferred_element_type=jnp.float32)
    wl = jax.nn.relu(wl.reshape(_T * _QROWS, 768) + bg_ref[...][None, :])
    g1 = jax.nn.relu(_dot_t(wl, w21_ref[...])
                     + b21_ref[...][None, :])
    g2 = jax.nn.relu(_dot_t(g1, w22_ref[...])
                     + b22_ref[...][None, :])
    out = _dot_t(g2, w23_ref[...]) + b23_ref[...][None, :]
    out_ref[...] = out.reshape(_T, _QROWS, 128)


def kernel(node_feat, adj, fc1_w1, fc1_b1, fc1_w2, fc1_b2, fc1_w3, fc1_b3,
           fc2_w1, fc2_b1, fc2_w2, fc2_b2, fc2_w3, fc2_b3, weight_G, bias_G):
    nt, ns, d = node_feat.shape
    nsup = adj.shape[1]
    del adj
    nq = ns - nsup
    nf = jnp.pad(node_feat, ((0, 0), (_SHIFT, _S - ns - _SHIFT), (0, 0)))
    qcat = node_feat[:, nsup:, :].reshape(nt, 1, nq * d)
    w23 = jnp.pad(fc2_w3, ((0, 128 - nq), (0, 0)))
    b23 = jnp.pad(fc2_b3, (0, 128 - nq))

    def task_map(t):
        return (t, 0, 0)

    consts = [fc1_w1, fc1_b1, fc1_w2, fc1_b2, fc1_w3, fc1_b3,
              weight_G, bias_G, fc2_w1, fc2_b1, fc2_w2, fc2_b2, w23, b23]

    def const_map_for(c):
        zeros = (0,) * c.ndim
        return lambda t: zeros

    out = pl.pallas_call(
        _meta_kernel,
        grid=(nt // _T,),
        in_specs=[pl.BlockSpec((_T, _S, d), task_map),
                  pl.BlockSpec((_T, 1, nq * d), task_map)]
                 + [pl.BlockSpec(c.shape, const_map_for(c)) for c in consts],
        out_specs=pl.BlockSpec((_T, _QROWS, 128), task_map),
        out_shape=jax.ShapeDtypeStruct((nt, _QROWS, 128), jnp.float32),
        compiler_params=pltpu.CompilerParams(
            dimension_semantics=("arbitrary",)),
    )(nf, qcat, *consts)
    return out[:, 0:nq, :nq]


# speedup vs baseline: 29.4108x; 1.1275x over previous
import jax
import jax.numpy as jnp
from jax.experimental import pallas as pl
from jax.experimental.pallas import tpu as pltpu

_S = 112
_NS = 105
_SHIFT = 4
_QROWS = 8
_T = 8

_DN_T = (((1,), (1,)), ((), ()))


def _dot_t(x, w):
    return jax.lax.dot_general(x, w, _DN_T,
                               preferred_element_type=jnp.float32)


def _meta_kernel(nf_ref, qcat_ref, w1_ref, b1_ref, w12_ref, b2_ref,
                 w13_ref, b3_ref, wg_ref, bg_ref,
                 w21_ref, b21_ref, w22_ref, b22_ref, w23_ref, b23_ref,
                 out_ref):
    nf3 = nf_ref[...]
    nf2 = nf3.reshape(_T * _S, 128)
    aq_in = nf3[:, _S - _QROWS:_S, :].reshape(_T * _QROWS, 128)
    a_q = _dot_t(aq_in, w1_ref[:, 0:128])
    b_all = _dot_t(nf2, w1_ref[:, 128:256])
    h1 = jax.nn.relu(a_q.reshape(_T, _QROWS, 1, 256)
                     + b_all.reshape(_T, 1, _S, 256)
                     + b1_ref[...][None, None, None, :])
    h2 = jax.nn.relu(_dot_t(h1.reshape(_T * _QROWS * _S, 256), w12_ref[...])
                     + b2_ref[...][None, :])
    s = jnp.sum(h2.reshape(_T, _QROWS, _S, 128)
                * w13_ref[...][None, None], axis=-1)
    s = jax.nn.sigmoid(s + b3_ref[0])
    col = jax.lax.broadcasted_iota(jnp.int32, (_T, _QROWS, _S), 2)
    la3 = jnp.where((col >= _SHIFT) & (col < _SHIFT + _NS), s, 0.0)
    sup = jnp.dot(nf2, wg_ref[0:128, :],
                  preferred_element_type=jnp.float32)
    qterm = jnp.dot(qcat_ref[...].reshape(_T, 640), wg_ref[128:768, :],
                    preferred_element_type=jnp.float32)
    sup3 = sup.reshape(_T, _S, 768) + qterm[:, None, :]
    wl = jax.lax.dot_general(la3, sup3, (((2,), (1,)), ((0,), (0,))),
                             preferred_element_type=jnp.float32)
    wl = jax.nn.relu(wl.reshape(_T * _QROWS, 768) + bg_ref[...][None, :])
    g1 = jax.nn.relu(_dot_t(wl, w21_ref[...])
                     + b21_ref[...][None, :])
    g2 = jax.nn.relu(_dot_t(g1, w22_ref[...])
                     + b22_ref[...][None, :])
    out = _dot_t(g2, w23_ref[...]) + b23_ref[...][None, :]
    out_ref[...] = out.reshape(_T, _QROWS, 128)


def kernel(node_feat, adj, fc1_w1, fc1_b1, fc1_w2, fc1_b2, fc1_w3, fc1_b3,
           fc2_w1, fc2_b1, fc2_w2, fc2_b2, fc2_w3, fc2_b3, weight_G, bias_G):
    nt, ns, d = node_feat.shape
    nsup = adj.shape[1]
    del adj
    nq = ns - nsup
    nf = jnp.pad(node_feat, ((0, 0), (_SHIFT, _S - ns - _SHIFT), (0, 0)))
    qcat = node_feat[:, nsup:, :].reshape(nt, 1, nq * d)
    w23 = jnp.pad(fc2_w3, ((0, 128 - nq), (0, 0)))
    b23 = jnp.pad(fc2_b3, (0, 128 - nq))

    def task_map(t):
        return (t, 0, 0)

    consts = [fc1_w1, fc1_b1, fc1_w2, fc1_b2, fc1_w3, fc1_b3,
              weight_G, bias_G, fc2_w1, fc2_b1, fc2_w2, fc2_b2, w23, b23]

    def const_map_for(c):
        zeros = (0,) * c.ndim
        return lambda t: zeros

    out = pl.pallas_call(
        _meta_kernel,
        grid=(nt // _T,),
        in_specs=[pl.BlockSpec((_T, _S, d), task_map),
                  pl.BlockSpec((_T, 1, nq * d), task_map)]
                 + [pl.BlockSpec(c.shape, const_map_for(c)) for c in consts],
        out_specs=pl.BlockSpec((_T, _QROWS, 128), task_map),
        out_shape=jax.ShapeDtypeStruct((nt, _QROWS, 128), jnp.float32),
        compiler_params=pltpu.CompilerParams(
            dimension_semantics=("arbitrary",)),
    )(nf, qcat, *consts)
    return out[:, 0:nq, :nq]


# speedup vs baseline: 30.7548x; 1.0457x over previous
import jax
import jax.numpy as jnp
from jax.experimental import pallas as pl
from jax.experimental.pallas import tpu as pltpu

_S = 112
_NS = 105
_SHIFT = 4
_QROWS = 8
_T = 16

_DN_T = (((1,), (1,)), ((), ()))


def _dot_t(x, w):
    return jax.lax.dot_general(x, w, _DN_T,
                               preferred_element_type=jnp.float32)


def _meta_kernel(nf_ref, qcat_ref, w1_ref, b1_ref, w12_ref, b2_ref,
                 w13_ref, b3_ref, wg_ref, bg_ref,
                 w21_ref, b21_ref, w22_ref, b22_ref, w23_ref, b23_ref,
                 out_ref):
    nf3 = nf_ref[...]
    nf2 = nf3.reshape(_T * _S, 128)
    aq_in = nf3[:, _S - _QROWS:_S, :].reshape(_T * _QROWS, 128)
    a_q = _dot_t(aq_in, w1_ref[:, 0:128])
    b_all = _dot_t(nf2, w1_ref[:, 128:256])
    h1 = jax.nn.relu(a_q.reshape(_T, _QROWS, 1, 256)
                     + b_all.reshape(_T, 1, _S, 256)
                     + b1_ref[...][None, None, None, :])
    h2 = jax.nn.relu(_dot_t(h1.reshape(_T * _QROWS * _S, 256), w12_ref[...])
                     + b2_ref[...][None, :])
    s = jnp.sum(h2.reshape(_T, _QROWS, _S, 128)
                * w13_ref[...][None, None], axis=-1)
    s = jax.nn.sigmoid(s + b3_ref[0])
    col = jax.lax.broadcasted_iota(jnp.int32, (_T, _QROWS, _S), 2)
    la3 = jnp.where((col >= _SHIFT) & (col < _SHIFT + _NS), s, 0.0)
    sup = jnp.dot(nf2, wg_ref[0:128, :],
                  preferred_element_type=jnp.float32)
    qterm = jnp.dot(qcat_ref[...].reshape(_T, 640), wg_ref[128:768, :],
                    preferred_element_type=jnp.float32)
    sup3 = sup.reshape(_T, _S, 768) + qterm[:, None, :]
    wl = jax.lax.dot_general(la3, sup3, (((2,), (1,)), ((0,), (0,))),
                             preferred_element_type=jnp.float32)
    wl = jax.nn.relu(wl.reshape(_T * _QROWS, 768) + bg_ref[...][None, :])
    g1 = jax.nn.relu(_dot_t(wl, w21_ref[...])
                     + b21_ref[...][None, :])
    g2 = jax.nn.relu(_dot_t(g1, w22_ref[...])
                     + b22_ref[...][None, :])
    out = _dot_t(g2, w23_ref[...]) + b23_ref[...][None, :]
    out_ref[...] = out.reshape(_T, _QROWS, 128)


def kernel(node_feat, adj, fc1_w1, fc1_b1, fc1_w2, fc1_b2, fc1_w3, fc1_b3,
           fc2_w1, fc2_b1, fc2_w2, fc2_b2, fc2_w3, fc2_b3, weight_G, bias_G):
    nt, ns, d = node_feat.shape
    nsup = adj.shape[1]
    del adj
    nq = ns - nsup
    nf = jnp.pad(node_feat, ((0, 0), (_SHIFT, _S - ns - _SHIFT), (0, 0)))
    qcat = node_feat[:, nsup:, :].reshape(nt, 1, nq * d)
    w23 = jnp.pad(fc2_w3, ((0, 128 - nq), (0, 0)))
    b23 = jnp.pad(fc2_b3, (0, 128 - nq))

    def task_map(t):
        return (t, 0, 0)

    consts = [fc1_w1, fc1_b1, fc1_w2, fc1_b2, fc1_w3, fc1_b3,
              weight_G, bias_G, fc2_w1, fc2_b1, fc2_w2, fc2_b2, w23, b23]

    def const_map_for(c):
        zeros = (0,) * c.ndim
        return lambda t: zeros

    out = pl.pallas_call(
        _meta_kernel,
        grid=(nt // _T,),
        in_specs=[pl.BlockSpec((_T, _S, d), task_map),
                  pl.BlockSpec((_T, 1, nq * d), task_map)]
                 + [pl.BlockSpec(c.shape, const_map_for(c)) for c in consts],
        out_specs=pl.BlockSpec((_T, _QROWS, 128), task_map),
        out_shape=jax.ShapeDtypeStruct((nt, _QROWS, 128), jnp.float32),
        compiler_params=pltpu.CompilerParams(
            dimension_semantics=("arbitrary",)),
    )(nf, qcat, *consts)
    return out[:, 0:nq, :nq]


# speedup vs baseline: 31.7040x; 1.0309x over previous
import jax
import jax.numpy as jnp
from jax.experimental import pallas as pl
from jax.experimental.pallas import tpu as pltpu

_S = 112
_NS = 105
_SHIFT = 4
_QROWS = 8
_T = 16

_DN_T = (((1,), (1,)), ((), ()))


def _dot_t(x, w):
    return jax.lax.dot_general(x, w, _DN_T,
                               preferred_element_type=jnp.float32)


def _meta_kernel(nf_ref, qcat_ref, w1_ref, b1_ref, w12_ref, b2_ref,
                 w13_ref, b3_ref, wg_ref, bg_ref,
                 w21_ref, b21_ref, w22_ref, b22_ref, w23_ref, b23_ref,
                 out_ref, s_scr):
    nf3 = nf_ref[...]
    nf2 = nf3.reshape(_T * _S, 128)
    aq_in = nf3[:, _S - _QROWS:_S, :].reshape(_T * _QROWS, 128)
    a_q = _dot_t(aq_in, w1_ref[:, 0:128]) + b1_ref[...][None, :]
    b_all = _dot_t(nf2, w1_ref[:, 128:256])
    h1 = jax.nn.relu(a_q.reshape(_T, _QROWS, 1, 256)
                     + b_all.reshape(_T, 1, _S, 256))
    h2 = jax.nn.relu(_dot_t(h1.reshape(_T * _QROWS * _S, 256), w12_ref[...])
                     + b2_ref[...][None, :])
    s_raw = jnp.sum(h2.reshape(_T, _QROWS, _S, 128)
                    * w13_ref[...][None, None], axis=-1)
    s_scr[...] = s_raw.reshape(_T * _QROWS, _S)
    s = jax.nn.sigmoid(s_scr[...].reshape(_T, _QROWS, _S) + b3_ref[0])
    col = jax.lax.broadcasted_iota(jnp.int32, (_T, _QROWS, _S), 2)
    la3 = jnp.where((col >= _SHIFT) & (col < _SHIFT + _NS), s, 0.0)
    sup = jnp.dot(nf2, wg_ref[0:128, :],
                  preferred_element_type=jnp.float32)
    qterm = jnp.dot(qcat_ref[...].reshape(_T, 640), wg_ref[128:768, :],
                    preferred_element_type=jnp.float32)
    sup3 = sup.reshape(_T, _S, 768) + qterm[:, None, :]
    wl = jax.lax.dot_general(la3, sup3, (((2,), (1,)), ((0,), (0,))),
                             preferred_element_type=jnp.float32)
    wl = jax.nn.relu(wl.reshape(_T * _QROWS, 768) + bg_ref[...][None, :])
    g1 = jax.nn.relu(_dot_t(wl, w21_ref[...])
                     + b21_ref[...][None, :])
    g2 = jax.nn.relu(_dot_t(g1, w22_ref[...])
                     + b22_ref[...][None, :])
    out = _dot_t(g2, w23_ref[...]) + b23_ref[...][None, :]
    out_ref[...] = out.reshape(_T, _QROWS, 128)


def kernel(node_feat, adj, fc1_w1, fc1_b1, fc1_w2, fc1_b2, fc1_w3, fc1_b3,
           fc2_w1, fc2_b1, fc2_w2, fc2_b2, fc2_w3, fc2_b3, weight_G, bias_G):
    nt, ns, d = node_feat.shape
    nsup = adj.shape[1]
    del adj
    nq = ns - nsup
    nf = jnp.pad(node_feat, ((0, 0), (_SHIFT, _S - ns - _SHIFT), (0, 0)))
    qcat = node_feat[:, nsup:, :].reshape(nt, 1, nq * d)
    w23 = jnp.pad(fc2_w3, ((0, 128 - nq), (0, 0)))
    b23 = jnp.pad(fc2_b3, (0, 128 - nq))

    def task_map(t):
        return (t, 0, 0)

    consts = [fc1_w1, fc1_b1, fc1_w2, fc1_b2, fc1_w3, fc1_b3,
              weight_G, bias_G, fc2_w1, fc2_b1, fc2_w2, fc2_b2, w23, b23]

    def const_map_for(c):
        zeros = (0,) * c.ndim
        return lambda t: zeros

    out = pl.pallas_call(
        _meta_kernel,
        grid=(nt // _T,),
        in_specs=[pl.BlockSpec((_T, _S, d), task_map),
                  pl.BlockSpec((_T, 1, nq * d), task_map)]
                 + [pl.BlockSpec(c.shape, const_map_for(c)) for c in consts],
        out_specs=pl.BlockSpec((_T, _QROWS, 128), task_map),
        out_shape=jax.ShapeDtypeStruct((nt, _QROWS, 128), jnp.float32),
        scratch_shapes=[pltpu.VMEM((_T * _QROWS, _S), jnp.float32)],
        compiler_params=pltpu.CompilerParams(
            dimension_semantics=("arbitrary",)),
    )(nf, qcat, *consts)
    return out[:, 0:nq, :nq]
